# baseline (speedup 1.0000x reference)
import jax
import jax.numpy as jnp
from jax import lax
from jax.experimental import pallas as pl
from jax.experimental.pallas import tpu as pltpu
from jax.experimental.pallas import tpu_sc as plsc

B = 16384
K = 5
V = 1000000
D = 64
NC = 2
NS = 16
L = 16
NW = NC * NS
BPW = B // NW
CHUNK = 128
NCHUNK = BPW // CHUNK
GATHER_ROWS = 128


def _preprocess_indices(src_hbm, start, count, row_v, half_v):
    pltpu.sync_copy(src_hbm.at[pl.ds(start, count)], row_v)

    def body(i, _):
        s = pl.ds(i * L, L)
        idx = row_v[s]
        half_v[s] = (idx & 1) * D
        row_v[s] = lax.shift_right_logical(idx, 1)
        return 0

    lax.fori_loop(0, count // L, body, 0)


def _sc_body(target_hbm, context_hbm, neg_hbm, in2_hbm, out2_hbm,
             pos_hbm, neg_out_hbm,
             row_t, half_t, row_c, half_c, row_n, half_n,
             tgt_v, ctx_v, neg_v, pos_v, negsc_v, sem):
    wid = lax.axis_index("s") * NC + lax.axis_index("c")
    base = wid * BPW
    lane = lax.iota(jnp.int32, L)

    for ci in range(NCHUNK):
        cbase = base + ci * CHUNK
        _preprocess_indices(target_hbm, cbase, CHUNK, row_t, half_t)
        _preprocess_indices(context_hbm, cbase, CHUNK, row_c, half_c)
        _preprocess_indices(neg_hbm, cbase * K, CHUNK * K, row_n, half_n)

        copies = []
        for j in range(CHUNK // GATHER_ROWS):
            s = pl.ds(j * GATHER_ROWS, GATHER_ROWS)
            copies.append(pltpu.async_copy(
                in2_hbm.at[row_t.at[s]], tgt_v.at[s], sem))
            copies.append(pltpu.async_copy(
                out2_hbm.at[row_c.at[s]], ctx_v.at[s], sem))
        for j in range(CHUNK * K // GATHER_ROWS):
            s = pl.ds(j * GATHER_ROWS, GATHER_ROWS)
            copies.append(pltpu.async_copy(
                out2_hbm.at[row_n.at[s]], neg_v.at[s], sem))
        for cp in copies:
            cp.wait()

        def group_body(g, _):
            rows_tc = g * L + lane
            colb_t = half_t[pl.ds(g * L, L)]
            colb_c = half_c[pl.ds(g * L, L)]
            colb_n = [plsc.load_gather(half_n, [rows_tc * K + k])
                      for k in range(K)]
            rows_n = [rows_tc * K + k for k in range(K)]

            def d_body(d, accs):
                acc_p = accs[0]
                t = plsc.load_gather(tgt_v, [rows_tc, colb_t + d])
                c = plsc.load_gather(ctx_v, [rows_tc, colb_c + d])
                acc_p = acc_p + t * c
                new_accs = [acc_p]
                for k in range(K):
                    n = plsc.load_gather(neg_v, [rows_n[k], colb_n[k] + d])
                    new_accs.append(accs[k + 1] + t * n)
                return tuple(new_accs)

            zeros = jnp.zeros((L,), jnp.float32)
            accs = lax.fori_loop(0, D, d_body, (zeros,) * (K + 1))

            off = ci * CHUNK + g * L
            plsc.store_scatter(pos_v, [off + lane], accs[0])
            for k in range(K):
                plsc.store_scatter(negsc_v, [(off + lane) * K + k],
                                   accs[k + 1])
            return 0

        lax.fori_loop(0, CHUNK // L, group_body, 0)

    pltpu.sync_copy(pos_v, pos_hbm.at[pl.ds(base, BPW)])
    pltpu.sync_copy(negsc_v, neg_out_hbm.at[pl.ds(base * K, BPW * K)])


_sc_scores = pl.kernel(
    _sc_body,
    out_type=(jax.ShapeDtypeStruct((B,), jnp.float32),
              jax.ShapeDtypeStruct((B * K,), jnp.float32)),
    mesh=plsc.VectorSubcoreMesh(core_axis_name="c", subcore_axis_name="s"),
    scratch_types=(
        pltpu.VMEM((CHUNK,), jnp.int32),
        pltpu.VMEM((CHUNK,), jnp.int32),
        pltpu.VMEM((CHUNK,), jnp.int32),
        pltpu.VMEM((CHUNK,), jnp.int32),
        pltpu.VMEM((CHUNK * K,), jnp.int32),
        pltpu.VMEM((CHUNK * K,), jnp.int32),
        pltpu.VMEM((CHUNK, 2 * D), jnp.float32),
        pltpu.VMEM((CHUNK, 2 * D), jnp.float32),
        pltpu.VMEM((CHUNK * K, 2 * D), jnp.float32),
        pltpu.VMEM((BPW,), jnp.float32),
        pltpu.VMEM((BPW * K,), jnp.float32),
        pltpu.SemaphoreType.DMA,
    ),
    compiler_params=pltpu.CompilerParams(needs_layout_passes=False),
)


def _loss_body(pos_ref, neg_ref, out_ref):
    lp = jnp.sum(jnp.log(jax.nn.sigmoid(pos_ref[...])))
    ln = jnp.sum(jnp.log(jax.nn.sigmoid(-neg_ref[...])))
    out_ref[0, 0] = -(lp / B + ln / (B * K))


_loss_kernel = pl.pallas_call(
    _loss_body,
    out_shape=jax.ShapeDtypeStruct((1, 1), jnp.float32),
    out_specs=pl.BlockSpec(memory_space=pltpu.SMEM),
)


@jax.jit
def kernel(target, context, neg_samples, in_embed, out_embed):
    pos_score, neg_score = _sc_scores(
        target.astype(jnp.int32), context.astype(jnp.int32),
        neg_samples.astype(jnp.int32),
        in_embed.reshape(V // 2, 2 * D), out_embed.reshape(V // 2, 2 * D))
    loss = _loss_kernel(pos_score.reshape(B // 128, 128),
                        neg_score.reshape(B * K // 128, 128))
    return loss[0, 0]


# speedup vs baseline: 1.2989x; 1.2989x over previous
import jax
import jax.numpy as jnp
from jax import lax
from jax.experimental import pallas as pl
from jax.experimental.pallas import tpu as pltpu
from jax.experimental.pallas import tpu_sc as plsc

B = 16384
K = 5
V = 1000000
D = 64
NC = 2
NS = 16
L = 16
NW = NC * NS
BPW = B // NW
CHUNK = 128
NCHUNK = BPW // CHUNK
GATHER_ROWS = 128

VB = 4096


def _pack_pairs(x):
    xt = x.T.reshape(VB // 2, 2, D)
    return jnp.concatenate([xt[:, 0, :], xt[:, 1, :]], axis=1)


def _transpose_body(inT_ref, outT_ref, in2_ref, out2_ref):
    in2_ref[...] = _pack_pairs(inT_ref[...])
    out2_ref[...] = _pack_pairs(outT_ref[...])


_transpose_tables = pl.pallas_call(
    _transpose_body,
    grid=(pl.cdiv(V, VB),),
    in_specs=[
        pl.BlockSpec((D, VB), lambda i: (0, i)),
        pl.BlockSpec((D, VB), lambda i: (0, i)),
    ],
    out_specs=[
        pl.BlockSpec((VB // 2, 2 * D), lambda i: (i, 0)),
        pl.BlockSpec((VB // 2, 2 * D), lambda i: (i, 0)),
    ],
    out_shape=[
        jax.ShapeDtypeStruct((V // 2, 2 * D), jnp.float32),
        jax.ShapeDtypeStruct((V // 2, 2 * D), jnp.float32),
    ],
)


def _preprocess_indices(src_hbm, start, count, row_v, half_v):
    pltpu.sync_copy(src_hbm.at[pl.ds(start, count)], row_v)

    def body(i, _):
        s = pl.ds(i * L, L)
        idx = row_v[s]
        half_v[s] = (idx & 1) * D
        row_v[s] = lax.shift_right_logical(idx, 1)
        return 0

    lax.fori_loop(0, count // L, body, 0)


def _sc_body(target_hbm, context_hbm, neg_hbm, in2_hbm, out2_hbm,
             pos_hbm, neg_out_hbm,
             row_t, half_t, row_c, half_c, row_n, half_n,
             tgt_v, ctx_v, neg_v, pos_v, negsc_v, sem):
    wid = lax.axis_index("s") * NC + lax.axis_index("c")
    base = wid * BPW
    lane = lax.iota(jnp.int32, L)

    for ci in range(NCHUNK):
        cbase = base + ci * CHUNK
        _preprocess_indices(target_hbm, cbase, CHUNK, row_t, half_t)
        _preprocess_indices(context_hbm, cbase, CHUNK, row_c, half_c)
        _preprocess_indices(neg_hbm, cbase * K, CHUNK * K, row_n, half_n)

        copies = []
        for j in range(CHUNK // GATHER_ROWS):
            s = pl.ds(j * GATHER_ROWS, GATHER_ROWS)
            copies.append(pltpu.async_copy(
                in2_hbm.at[row_t.at[s]], tgt_v.at[s], sem))
            copies.append(pltpu.async_copy(
                out2_hbm.at[row_c.at[s]], ctx_v.at[s], sem))
        for j in range(CHUNK * K // GATHER_ROWS):
            s = pl.ds(j * GATHER_ROWS, GATHER_ROWS)
            copies.append(pltpu.async_copy(
                out2_hbm.at[row_n.at[s]], neg_v.at[s], sem))
        for cp in copies:
            cp.wait()

        def group_body(g, _):
            rows_tc = g * L + lane
            colb_t = half_t[pl.ds(g * L, L)]
            colb_c = half_c[pl.ds(g * L, L)]
            colb_n = [plsc.load_gather(half_n, [rows_tc * K + k])
                      for k in range(K)]
            rows_n = [rows_tc * K + k for k in range(K)]

            def d_body(d, accs):
                acc_p = accs[0]
                t = plsc.load_gather(tgt_v, [rows_tc, colb_t + d])
                c = plsc.load_gather(ctx_v, [rows_tc, colb_c + d])
                acc_p = acc_p + t * c
                new_accs = [acc_p]
                for k in range(K):
                    n = plsc.load_gather(neg_v, [rows_n[k], colb_n[k] + d])
                    new_accs.append(accs[k + 1] + t * n)
                return tuple(new_accs)

            zeros = jnp.zeros((L,), jnp.float32)
            accs = lax.fori_loop(0, D, d_body, (zeros,) * (K + 1))

            off = ci * CHUNK + g * L
            plsc.store_scatter(pos_v, [off + lane], accs[0])
            for k in range(K):
                plsc.store_scatter(negsc_v, [(off + lane) * K + k],
                                   accs[k + 1])
            return 0

        lax.fori_loop(0, CHUNK // L, group_body, 0)

    pltpu.sync_copy(pos_v, pos_hbm.at[pl.ds(base, BPW)])
    pltpu.sync_copy(negsc_v, neg_out_hbm.at[pl.ds(base * K, BPW * K)])


_sc_scores = pl.kernel(
    _sc_body,
    out_type=(jax.ShapeDtypeStruct((B,), jnp.float32),
              jax.ShapeDtypeStruct((B * K,), jnp.float32)),
    mesh=plsc.VectorSubcoreMesh(core_axis_name="c", subcore_axis_name="s"),
    scratch_types=(
        pltpu.VMEM((CHUNK,), jnp.int32),
        pltpu.VMEM((CHUNK,), jnp.int32),
        pltpu.VMEM((CHUNK,), jnp.int32),
        pltpu.VMEM((CHUNK,), jnp.int32),
        pltpu.VMEM((CHUNK * K,), jnp.int32),
        pltpu.VMEM((CHUNK * K,), jnp.int32),
        pltpu.VMEM((CHUNK, 2 * D), jnp.float32),
        pltpu.VMEM((CHUNK, 2 * D), jnp.float32),
        pltpu.VMEM((CHUNK * K, 2 * D), jnp.float32),
        pltpu.VMEM((BPW,), jnp.float32),
        pltpu.VMEM((BPW * K,), jnp.float32),
        pltpu.SemaphoreType.DMA,
    ),
    compiler_params=pltpu.CompilerParams(needs_layout_passes=False),
)


def _loss_body(pos_ref, neg_ref, out_ref):
    lp = jnp.sum(jnp.log(jax.nn.sigmoid(pos_ref[...])))
    ln = jnp.sum(jnp.log(jax.nn.sigmoid(-neg_ref[...])))
    out_ref[0, 0] = -(lp / B + ln / (B * K))


_loss_kernel = pl.pallas_call(
    _loss_body,
    out_shape=jax.ShapeDtypeStruct((1, 1), jnp.float32),
    out_specs=pl.BlockSpec(memory_space=pltpu.SMEM),
)


@jax.jit
def kernel(target, context, neg_samples, in_embed, out_embed):
    in2, out2 = _transpose_tables(in_embed.T, out_embed.T)
    pos_score, neg_score = _sc_scores(
        target.astype(jnp.int32), context.astype(jnp.int32),
        neg_samples.astype(jnp.int32), in2, out2)
    loss = _loss_kernel(pos_score.reshape(B // 128, 128),
                        neg_score.reshape(B * K // 128, 128))
    return loss[0, 0]
